# Initial kernel scaffold; baseline (speedup 1.0000x reference)
import jax
import jax.numpy as jnp
from jax.experimental import pallas as pl


def kernel(x, edge_index, W1, bn1_gamma, bn1_beta, W2, bn2_gamma, bn2_beta):
    raise NotImplementedError("write your pallas kernel here")


# speedup vs baseline: 3.7276x; 3.7276x over previous
import functools

import jax
import jax.numpy as jnp
from jax import lax
from jax.experimental import pallas as pl
from jax.experimental.pallas import tpu as pltpu
from jax.experimental.pallas import tpu_sc as plsc

_N = 10000
_E = 160000
_D = 256
_H = 256
_EPS = 1e-5

_RB = 400
_NRB = _N // _RB

_NSUB = 16
_B = 128
_NB = -(-_E // (_NSUB * _B))
_CH = _NB * _B
_EPAD = _NSUB * _CH
_RPT = 640
_ACC_ROWS = _NSUB * _RPT


def _mm_stats_body(x_ref, w_ref, h1_ref, stats_ref, acc_ref):
    i = pl.program_id(0)
    h1 = jnp.dot(x_ref[...], w_ref[...], preferred_element_type=jnp.float32)
    h1_ref[...] = h1
    s = jnp.sum(h1, axis=0, keepdims=True)
    s2 = jnp.sum(h1 * h1, axis=0, keepdims=True)
    ss = jnp.concatenate([s, s2], axis=0)

    @pl.when(i == 0)
    def _():
        acc_ref[...] = jnp.zeros_like(acc_ref)

    acc_ref[...] += ss

    @pl.when(i == pl.num_programs(0) - 1)
    def _():
        stats_ref[...] = acc_ref[...]


_mm_stats = pl.pallas_call(
    _mm_stats_body,
    grid=(_NRB,),
    in_specs=[
        pl.BlockSpec((_RB, _D), lambda i: (i, 0)),
        pl.BlockSpec((_D, _H), lambda i: (0, 0)),
    ],
    out_specs=[
        pl.BlockSpec((_RB, _H), lambda i: (i, 0)),
        pl.BlockSpec((2, _H), lambda i: (0, 0)),
    ],
    out_shape=[
        jax.ShapeDtypeStruct((_N, _H), jnp.float32),
        jax.ShapeDtypeStruct((2, _H), jnp.float32),
    ],
    scratch_shapes=[pltpu.VMEM((2, _H), jnp.float32)],
)


def _mm2_body(h1_ref, ab_ref, w_ref, z_ref):
    h1s = h1_ref[...] * ab_ref[0:1, :] + ab_ref[1:2, :]
    z = jnp.dot(h1s, w_ref[...], preferred_element_type=jnp.float32)
    z_ref[0] = z[:, 0:128]
    z_ref[1] = z[:, 128:256]


_mm2 = pl.pallas_call(
    _mm2_body,
    grid=(_NRB,),
    in_specs=[
        pl.BlockSpec((_RB, _H), lambda i: (i, 0)),
        pl.BlockSpec((2, _H), lambda i: (0, 0)),
        pl.BlockSpec((_H, _D), lambda i: (0, 0)),
    ],
    out_specs=pl.BlockSpec((2, _RB, 128), lambda i: (0, i, 0)),
    out_shape=jax.ShapeDtypeStruct((2, _N, 128), jnp.float32),
)


def _stats2_body(h_ref, stats_ref, acc_ref):
    i = pl.program_id(0)
    hb = h_ref[...]
    s = jnp.sum(hb, axis=1)
    s2 = jnp.sum(hb * hb, axis=1)
    ss = jnp.stack([s, s2], axis=0)

    @pl.when(i == 0)
    def _():
        acc_ref[...] = jnp.zeros_like(acc_ref)

    acc_ref[...] += ss

    @pl.when(i == pl.num_programs(0) - 1)
    def _():
        stats_ref[...] = acc_ref[...]


_stats2 = pl.pallas_call(
    _stats2_body,
    grid=(_NRB,),
    in_specs=[pl.BlockSpec((2, _RB, 128), lambda i: (0, i, 0))],
    out_specs=pl.BlockSpec((2, 2, 128), lambda i: (0, 0, 0)),
    out_shape=jax.ShapeDtypeStruct((2, 2, 128), jnp.float32),
    scratch_shapes=[pltpu.VMEM((2, 2, 128), jnp.float32)],
)


def _bn2_body(h_ref, ab_ref, out_ref):
    hb = h_ref[...]
    a = ab_ref[0]
    b = ab_ref[1]
    y = hb * a[:, None, :] + b[:, None, :]
    out_ref[:, 0:128] = y[0]
    out_ref[:, 128:256] = y[1]


_bn2 = pl.pallas_call(
    _bn2_body,
    grid=(_NRB,),
    in_specs=[
        pl.BlockSpec((2, _RB, 128), lambda i: (0, i, 0)),
        pl.BlockSpec((2, 2, 128), lambda i: (0, 0, 0)),
    ],
    out_specs=pl.BlockSpec((_RB, _D), lambda i: (i, 0)),
    out_shape=jax.ShapeDtypeStruct((_N, _D), jnp.float32),
)


def _seg_body(zf_h, src0_h, src1_h, dst_h, zrows_h, out_h,
              src_v, dst_v, rows_v, acc_s, sem):
    c = lax.axis_index("c")
    s = lax.axis_index("s")
    r0 = s * _RPT

    pltpu.sync_copy(zrows_h, acc_s.at[pl.ds(r0, _RPT)])

    base = s * _CH

    @pl.when(c == 0)
    def _():
        pltpu.sync_copy(src0_h.at[pl.ds(base, _CH)], src_v)

    @pl.when(c == 1)
    def _():
        pltpu.sync_copy(src1_h.at[pl.ds(base, _CH)], src_v)

    pltpu.sync_copy(dst_h.at[s], dst_v)
    plsc.subcore_barrier()

    def body(j, carry):
        off = pl.multiple_of(j * _B, _B)
        pltpu.async_copy(zf_h.at[src_v.at[pl.ds(off, _B)]], rows_v, sem).wait()
        pltpu.sync_copy(rows_v, acc_s.at[dst_v.at[j]], add=True)
        return carry

    lax.fori_loop(0, _NB, body, 0)
    plsc.subcore_barrier()

    out_base = c * _N + r0

    @pl.when(s < _NSUB - 1)
    def _():
        pltpu.sync_copy(acc_s.at[pl.ds(r0, _RPT)], out_h.at[pl.ds(out_base, _RPT)])

    @pl.when(s == _NSUB - 1)
    def _():
        pltpu.sync_copy(acc_s.at[pl.ds(r0, _N - (_NSUB - 1) * _RPT)],
                        out_h.at[pl.ds(out_base, _N - (_NSUB - 1) * _RPT)])


_seg_sum = functools.partial(
    pl.kernel,
    mesh=plsc.VectorSubcoreMesh(core_axis_name="c", subcore_axis_name="s"),
    out_type=jax.ShapeDtypeStruct((2 * _N, 128), jnp.float32),
    scratch_types=[
        pltpu.VMEM((_CH,), jnp.int32),
        pltpu.VMEM((_NB, _B), jnp.int32),
        pltpu.VMEM((_B, 128), jnp.float32),
        pltpu.VMEM_SHARED((_ACC_ROWS, 128), jnp.float32),
        pltpu.SemaphoreType.DMA,
    ],
)(_seg_body)


def kernel(x, edge_index, W1, bn1_gamma, bn1_beta, W2, bn2_gamma, bn2_beta):
    h1, st1 = _mm_stats(x, W1)
    mean1 = st1[0] / _N
    var1 = st1[1] / _N - mean1 * mean1
    a1 = bn1_gamma / jnp.sqrt(var1 + _EPS)
    b1 = bn1_beta - mean1 * a1
    ab1 = jnp.stack([a1, b1])

    z = _mm2(h1, ab1, W2)

    src = edge_index[0]
    dst = edge_index[1]
    pad = _EPAD - _E
    src_p = jnp.concatenate([src, jnp.zeros((pad,), jnp.int32)])
    dst_p = jnp.concatenate([dst, jnp.full((pad,), _N, jnp.int32)])
    dst_p = dst_p.reshape(_NSUB, _NB, _B)
    zf = z.reshape(2 * _N, 128)
    zrows = jnp.zeros((_RPT, 128), jnp.float32)

    hf = _seg_sum(zf, src_p, src_p + _N, dst_p, zrows)
    h2 = hf.reshape(2, _N, 128)

    st2 = _stats2(h2)
    mean2 = st2[0] / _N
    var2 = st2[1] / _N - mean2 * mean2
    a2 = bn2_gamma.reshape(2, 128) / jnp.sqrt(var2 + _EPS)
    b2 = bn2_beta.reshape(2, 128) - mean2 * a2
    ab2 = jnp.stack([a2, b2])

    return _bn2(h2, ab2)
